# Initial kernel scaffold; baseline (speedup 1.0000x reference)
import jax
import jax.numpy as jnp
from jax.experimental import pallas as pl


def kernel(x):
    raise NotImplementedError("write your pallas kernel here")


# speedup vs baseline: 5.2880x; 5.2880x over previous
import jax
import jax.numpy as jnp
from jax.experimental import pallas as pl
from jax.experimental.pallas import tpu as pltpu

K = 64
N_ROWS = 131072
N_COLS = 128
BLOCK_ROWS = 1024
SUB = 64
GRID = N_ROWS // BLOCK_ROWS
NSUB = BLOCK_ROWS // SUB


def _bit_f32(shape, j):
    i = jax.lax.broadcasted_iota(jnp.int32, shape, 0)
    return ((i >> (j.bit_length() - 1)) & 1).astype(jnp.float32)


def _masks(shape):
    out = {}
    for j in (1, 2, 4, 8, 16, 32):
        hi = _bit_f32(shape, j)
        out[j] = (1.0 - hi, hi)
    return out


def _stage_u(y, j, mlow):
    lo_m, hi_m = mlow
    up = jnp.roll(y, -j, axis=0)
    dn = jnp.roll(y, j, axis=0)
    max_lo = jnp.maximum(y, up)
    min_hi = jnp.minimum(y, dn)
    return lo_m * max_lo + hi_m * min_hi


def _sort_asc(x, mlow, signs):
    n = x.shape[0]
    y = signs[2] * x
    k = 2
    while k <= n:
        if k > 2:
            y = y * (signs[k] * signs[k // 2])
        j = k // 2
        while j >= 1:
            y = _stage_u(y, j, mlow[j])
            j //= 2
        k *= 2
    return -y


def _merge_desc_topk(r, b_asc, mlow):
    h = jnp.maximum(r, b_asc)
    j = h.shape[0] // 2
    while j >= 1:
        h = _stage_u(h, j, mlow[j])
        j //= 2
    return h


def _sign_arrays(shape, n):
    s = {}
    k = 2
    while k < n:
        s[k] = 2.0 * _bit_f32(shape, k) - 1.0
        k *= 2
    s[n] = jnp.full(shape, -1.0, jnp.float32)
    return s


def _topk_kernel(x_ref, o_ref, r_ref):
    step = pl.program_id(0)

    @pl.when(step == 0)
    def _init():
        r_ref[...] = jnp.full((K, N_COLS), -jnp.inf, jnp.float32)

    shape = (SUB, N_COLS)
    mlow = _masks(shape)
    signs = _sign_arrays(shape, SUB)

    r = r_ref[...]
    for s in range(NSUB):
        blk = x_ref[s * SUB:(s + 1) * SUB, :]
        b = _sort_asc(blk, mlow, signs)
        r = _merge_desc_topk(r, b, mlow)
    r_ref[...] = r

    @pl.when(step == GRID - 1)
    def _out():
        o_ref[...] = r_ref[...]


@jax.jit
def kernel(x):
    out = pl.pallas_call(
        _topk_kernel,
        grid=(GRID,),
        in_specs=[pl.BlockSpec((BLOCK_ROWS, N_COLS), lambda i: (i, 0))],
        out_specs=pl.BlockSpec((K, N_COLS), lambda i: (0, 0)),
        out_shape=jax.ShapeDtypeStruct((K, N_COLS), jnp.float32),
        scratch_shapes=[pltpu.VMEM((K, N_COLS), jnp.float32)],
    )(x)
    return out.reshape(-1)


# speedup vs baseline: 14.1230x; 2.6708x over previous
import jax
import jax.numpy as jnp
from jax.experimental import pallas as pl
from jax.experimental.pallas import tpu as pltpu

K = 64
N_ROWS = 131072
N_COLS = 128
BLOCK_ROWS = 512
GRID = N_ROWS // BLOCK_ROWS
NPOS = 64


def _comparator(a, mn_i, mx_i):
    lo = jnp.minimum(a[mn_i], a[mx_i])
    hi = jnp.maximum(a[mn_i], a[mx_i])
    a[mn_i], a[mx_i] = lo, hi


def _sort_positions_asc(a):
    n = len(a)
    k = 2
    while k <= n:
        j = k // 2
        while j >= 1:
            for p in range(n):
                if p & j == 0:
                    q = p | j
                    if p & k == 0 or k == n:
                        _comparator(a, p, q)
                    else:
                        _comparator(a, q, p)
            j //= 2
        k *= 2


def _merge_into_r(r, a):
    h = [jnp.maximum(r[p], a[p]) for p in range(NPOS)]
    j = NPOS // 2
    while j >= 1:
        for p in range(NPOS):
            if p & j == 0:
                _comparator(h, p | j, p)
        j //= 2
    for p in range(NPOS):
        r[p] = h[p]


def _bit_f32(shape, j):
    i = jax.lax.broadcasted_iota(jnp.int32, shape, 0)
    return ((i >> (j.bit_length() - 1)) & 1).astype(jnp.float32)


def _stage_u(y, j, lo_m, hi_m):
    up = jnp.roll(y, -j, axis=0)
    dn = jnp.roll(y, j, axis=0)
    max_lo = jnp.maximum(y, up)
    min_hi = jnp.minimum(y, dn)
    return lo_m * max_lo + hi_m * min_hi


def _sort_desc_full(x):
    n = x.shape[0]
    shape = x.shape
    masks = {}
    j = 1
    while j < n:
        hi = _bit_f32(shape, j)
        masks[j] = (1.0 - hi, hi)
        j *= 2
    signs = {}
    k = 2
    while k < n:
        signs[k] = 1.0 - 2.0 * _bit_f32(shape, k)
        k *= 2
    signs[n] = jnp.full(shape, 1.0, jnp.float32)

    y = signs[2] * x
    k = 2
    while k <= n:
        if k > 2:
            y = y * (signs[k] * signs[k // 2])
        j = k // 2
        while j >= 1:
            y = _stage_u(y, j, *masks[j])
            j //= 2
        k *= 2
    return y


def _topk_kernel(x_ref, o_ref, r_ref):
    step = pl.program_id(0)

    @pl.when(step == 0)
    def _init():
        r_ref[...] = jnp.full((BLOCK_ROWS, N_COLS), -jnp.inf, jnp.float32)

    a = [x_ref[8 * p:8 * (p + 1), :] for p in range(NPOS)]
    _sort_positions_asc(a)
    r = [r_ref[8 * p:8 * (p + 1), :] for p in range(NPOS)]
    _merge_into_r(r, a)
    for p in range(NPOS):
        r_ref[8 * p:8 * (p + 1), :] = r[p]

    @pl.when(step == GRID - 1)
    def _out():
        o_ref[...] = _sort_desc_full(r_ref[...])[:K, :]


@jax.jit
def kernel(x):
    out = pl.pallas_call(
        _topk_kernel,
        grid=(GRID,),
        in_specs=[pl.BlockSpec((BLOCK_ROWS, N_COLS), lambda i: (i, 0))],
        out_specs=pl.BlockSpec((K, N_COLS), lambda i: (0, 0)),
        out_shape=jax.ShapeDtypeStruct((K, N_COLS), jnp.float32),
        scratch_shapes=[pltpu.VMEM((BLOCK_ROWS, N_COLS), jnp.float32)],
    )(x)
    return out.reshape(-1)


# speedup vs baseline: 14.6231x; 1.0354x over previous
import jax
import jax.numpy as jnp
from jax.experimental import pallas as pl
from jax.experimental.pallas import tpu as pltpu

K = 64
N_ROWS = 131072
N_COLS = 128
BLOCK_ROWS = 512
GRID = N_ROWS // BLOCK_ROWS
NPOS = 64


def _comparator(a, mn_i, mx_i):
    lo = jnp.minimum(a[mn_i], a[mx_i])
    hi = jnp.maximum(a[mn_i], a[mx_i])
    a[mn_i], a[mx_i] = lo, hi


def _oddeven_merge(lo, hi, r):
    step = r * 2
    if step < hi - lo:
        yield from _oddeven_merge(lo, hi, step)
        yield from _oddeven_merge(lo + r, hi, step)
        yield from ((i, i + r) for i in range(lo + r, hi - r, step))
    else:
        yield (lo, lo + r)


def _oddeven_sort_pairs(lo, hi):
    if hi - lo >= 1:
        mid = lo + (hi - lo) // 2
        yield from _oddeven_sort_pairs(lo, mid)
        yield from _oddeven_sort_pairs(mid + 1, hi)
        yield from _oddeven_merge(lo, hi, 1)


def _sort_positions_asc(a):
    for p, q in _oddeven_sort_pairs(0, len(a) - 1):
        _comparator(a, p, q)


def _merge_into_r(r, a):
    h = [jnp.maximum(r[p], a[p]) for p in range(NPOS)]
    j = NPOS // 2
    while j >= 1:
        for p in range(NPOS):
            if p & j == 0:
                _comparator(h, p | j, p)
        j //= 2
    for p in range(NPOS):
        r[p] = h[p]


def _bit_f32(shape, j):
    i = jax.lax.broadcasted_iota(jnp.int32, shape, 0)
    return ((i >> (j.bit_length() - 1)) & 1).astype(jnp.float32)


def _stage_u(y, j, lo_m, hi_m):
    up = jnp.roll(y, -j, axis=0)
    dn = jnp.roll(y, j, axis=0)
    max_lo = jnp.maximum(y, up)
    min_hi = jnp.minimum(y, dn)
    return lo_m * max_lo + hi_m * min_hi


def _sort_desc_full(x):
    n = x.shape[0]
    shape = x.shape
    masks = {}
    j = 1
    while j < n:
        hi = _bit_f32(shape, j)
        masks[j] = (1.0 - hi, hi)
        j *= 2
    signs = {}
    k = 2
    while k < n:
        signs[k] = 1.0 - 2.0 * _bit_f32(shape, k)
        k *= 2
    signs[n] = jnp.full(shape, 1.0, jnp.float32)

    y = signs[2] * x
    k = 2
    while k <= n:
        if k > 2:
            y = y * (signs[k] * signs[k // 2])
        j = k // 2
        while j >= 1:
            y = _stage_u(y, j, *masks[j])
            j //= 2
        k *= 2
    return y


def _topk_kernel(x_ref, o_ref, r_ref):
    step = pl.program_id(0)

    @pl.when(step == 0)
    def _init():
        r_ref[...] = jnp.full((BLOCK_ROWS, N_COLS), -jnp.inf, jnp.float32)

    a = [x_ref[8 * p:8 * (p + 1), :] for p in range(NPOS)]
    _sort_positions_asc(a)
    r = [r_ref[8 * p:8 * (p + 1), :] for p in range(NPOS)]
    _merge_into_r(r, a)
    for p in range(NPOS):
        r_ref[8 * p:8 * (p + 1), :] = r[p]

    @pl.when(step == GRID - 1)
    def _out():
        o_ref[...] = _sort_desc_full(r_ref[...])[:K, :]


@jax.jit
def kernel(x):
    out = pl.pallas_call(
        _topk_kernel,
        grid=(GRID,),
        in_specs=[pl.BlockSpec((BLOCK_ROWS, N_COLS), lambda i: (i, 0))],
        out_specs=pl.BlockSpec((K, N_COLS), lambda i: (0, 0)),
        out_shape=jax.ShapeDtypeStruct((K, N_COLS), jnp.float32),
        scratch_shapes=[pltpu.VMEM((BLOCK_ROWS, N_COLS), jnp.float32)],
    )(x)
    return out.reshape(-1)


# speedup vs baseline: 31.4963x; 2.1539x over previous
import jax
import jax.numpy as jnp
from jax.experimental import pallas as pl
from jax.experimental.pallas import tpu as pltpu

K = 64
N_ROWS = 131072
N_COLS = 128
NPOS = 64
SET_ROWS = 8 * NPOS
NSETS = 4
BLOCK_ROWS = SET_ROWS * NSETS
GRID = N_ROWS // BLOCK_ROWS


def _comparator(a, mn_i, mx_i):
    lo = jnp.minimum(a[mn_i], a[mx_i])
    hi = jnp.maximum(a[mn_i], a[mx_i])
    a[mn_i], a[mx_i] = lo, hi


def _oddeven_merge(lo, hi, r):
    step = r * 2
    if step < hi - lo:
        yield from _oddeven_merge(lo, hi, step)
        yield from _oddeven_merge(lo + r, hi, step)
        yield from ((i, i + r) for i in range(lo + r, hi - r, step))
    else:
        yield (lo, lo + r)


def _oddeven_sort_pairs(lo, hi):
    if hi - lo >= 1:
        mid = lo + (hi - lo) // 2
        yield from _oddeven_sort_pairs(lo, mid)
        yield from _oddeven_sort_pairs(mid + 1, hi)
        yield from _oddeven_merge(lo, hi, 1)


def _sort_positions_asc(a):
    for p, q in _oddeven_sort_pairs(0, len(a) - 1):
        _comparator(a, p, q)


def _merge_into_r(r, a):
    h = [jnp.maximum(r[p], a[p]) for p in range(NPOS)]
    j = NPOS // 2
    while j >= 1:
        for p in range(NPOS):
            if p & j == 0:
                _comparator(h, p | j, p)
        j //= 2
    for p in range(NPOS):
        r[p] = h[p]


def _bit_f32(shape, j):
    i = jax.lax.broadcasted_iota(jnp.int32, shape, 0)
    return ((i >> (j.bit_length() - 1)) & 1).astype(jnp.float32)


def _stage_u(y, j, lo_m, hi_m):
    up = jnp.roll(y, -j, axis=0)
    dn = jnp.roll(y, j, axis=0)
    max_lo = jnp.maximum(y, up)
    min_hi = jnp.minimum(y, dn)
    return lo_m * max_lo + hi_m * min_hi


def _sort_desc_full(x):
    n = x.shape[0]
    shape = x.shape
    masks = {}
    j = 1
    while j < n:
        hi = _bit_f32(shape, j)
        masks[j] = (1.0 - hi, hi)
        j *= 2
    signs = {}
    k = 2
    while k < n:
        signs[k] = 1.0 - 2.0 * _bit_f32(shape, k)
        k *= 2
    signs[n] = jnp.full(shape, 1.0, jnp.float32)

    y = signs[2] * x
    k = 2
    while k <= n:
        if k > 2:
            y = y * (signs[k] * signs[k // 2])
        j = k // 2
        while j >= 1:
            y = _stage_u(y, j, *masks[j])
            j //= 2
        k *= 2
    return y


def _topk_kernel(x_ref, o_ref, r_ref):
    step = pl.program_id(0)

    @pl.when(step == 0)
    def _init():
        r_ref[...] = jnp.full((SET_ROWS, N_COLS), -jnp.inf, jnp.float32)

    r = [r_ref[8 * p:8 * (p + 1), :] for p in range(NPOS)]
    for s in range(NSETS):
        base = s * SET_ROWS
        a = [x_ref[base + 8 * p:base + 8 * (p + 1), :] for p in range(NPOS)]
        _sort_positions_asc(a)
        _merge_into_r(r, a)
    for p in range(NPOS):
        r_ref[8 * p:8 * (p + 1), :] = r[p]

    @pl.when(step == GRID - 1)
    def _out():
        o_ref[...] = _sort_desc_full(r_ref[...])[:K, :]


@jax.jit
def kernel(x):
    out = pl.pallas_call(
        _topk_kernel,
        grid=(GRID,),
        in_specs=[pl.BlockSpec((BLOCK_ROWS, N_COLS), lambda i: (i, 0))],
        out_specs=pl.BlockSpec((K, N_COLS), lambda i: (0, 0)),
        out_shape=jax.ShapeDtypeStruct((K, N_COLS), jnp.float32),
        scratch_shapes=[pltpu.VMEM((SET_ROWS, N_COLS), jnp.float32)],
    )(x)
    return out.reshape(-1)


# speedup vs baseline: 36.8785x; 1.1709x over previous
import jax
import jax.numpy as jnp
from jax.experimental import pallas as pl
from jax.experimental.pallas import tpu as pltpu

K = 64
N_ROWS = 131072
N_COLS = 128
NPOS = 64
SET_ROWS = 8 * NPOS
NSETS = 8
BLOCK_ROWS = SET_ROWS * NSETS
GRID = N_ROWS // BLOCK_ROWS


def _comparator(a, mn_i, mx_i):
    lo = jnp.minimum(a[mn_i], a[mx_i])
    hi = jnp.maximum(a[mn_i], a[mx_i])
    a[mn_i], a[mx_i] = lo, hi


def _oddeven_merge(lo, hi, r):
    step = r * 2
    if step < hi - lo:
        yield from _oddeven_merge(lo, hi, step)
        yield from _oddeven_merge(lo + r, hi, step)
        yield from ((i, i + r) for i in range(lo + r, hi - r, step))
    else:
        yield (lo, lo + r)


def _oddeven_sort_pairs(lo, hi):
    if hi - lo >= 1:
        mid = lo + (hi - lo) // 2
        yield from _oddeven_sort_pairs(lo, mid)
        yield from _oddeven_sort_pairs(mid + 1, hi)
        yield from _oddeven_merge(lo, hi, 1)


def _sort_positions_asc(a):
    for p, q in _oddeven_sort_pairs(0, len(a) - 1):
        _comparator(a, p, q)


def _merge_into_r(r, a):
    h = [jnp.maximum(r[p], a[p]) for p in range(NPOS)]
    j = NPOS // 2
    while j >= 1:
        for p in range(NPOS):
            if p & j == 0:
                _comparator(h, p | j, p)
        j //= 2
    for p in range(NPOS):
        r[p] = h[p]


def _bit_f32(shape, j):
    i = jax.lax.broadcasted_iota(jnp.int32, shape, 0)
    return ((i >> (j.bit_length() - 1)) & 1).astype(jnp.float32)


def _stage_u(y, j, lo_m, hi_m):
    up = jnp.roll(y, -j, axis=0)
    dn = jnp.roll(y, j, axis=0)
    max_lo = jnp.maximum(y, up)
    min_hi = jnp.minimum(y, dn)
    return lo_m * max_lo + hi_m * min_hi


def _sort_desc_full(x):
    n = x.shape[0]
    shape = x.shape
    masks = {}
    j = 1
    while j < n:
        hi = _bit_f32(shape, j)
        masks[j] = (1.0 - hi, hi)
        j *= 2
    signs = {}
    k = 2
    while k < n:
        signs[k] = 1.0 - 2.0 * _bit_f32(shape, k)
        k *= 2
    signs[n] = jnp.full(shape, 1.0, jnp.float32)

    y = signs[2] * x
    k = 2
    while k <= n:
        if k > 2:
            y = y * (signs[k] * signs[k // 2])
        j = k // 2
        while j >= 1:
            y = _stage_u(y, j, *masks[j])
            j //= 2
        k *= 2
    return y


def _topk_kernel(x_ref, o_ref, r_ref):
    step = pl.program_id(0)

    @pl.when(step == 0)
    def _init():
        r_ref[...] = jnp.full((SET_ROWS, N_COLS), -jnp.inf, jnp.float32)

    r = [r_ref[8 * p:8 * (p + 1), :] for p in range(NPOS)]
    for s in range(NSETS):
        base = s * SET_ROWS
        a = [x_ref[base + 8 * p:base + 8 * (p + 1), :] for p in range(NPOS)]
        _sort_positions_asc(a)
        _merge_into_r(r, a)
    for p in range(NPOS):
        r_ref[8 * p:8 * (p + 1), :] = r[p]

    @pl.when(step == GRID - 1)
    def _out():
        o_ref[...] = _sort_desc_full(r_ref[...])[:K, :]


@jax.jit
def kernel(x):
    out = pl.pallas_call(
        _topk_kernel,
        grid=(GRID,),
        in_specs=[pl.BlockSpec((BLOCK_ROWS, N_COLS), lambda i: (i, 0))],
        out_specs=pl.BlockSpec((K, N_COLS), lambda i: (0, 0)),
        out_shape=jax.ShapeDtypeStruct((K, N_COLS), jnp.float32),
        scratch_shapes=[pltpu.VMEM((SET_ROWS, N_COLS), jnp.float32)],
    )(x)
    return out.reshape(-1)
